# baseline (speedup 1.0000x reference)
import functools

import jax
import jax.numpy as jnp
from jax import lax
from jax.experimental import pallas as pl
from jax.experimental.pallas import tpu as pltpu
from jax.experimental.pallas import tpu_sc as plsc

E = 8
D_IN = 1024
D_OUT = 1024
D_PROJ = 256
N_TOK = 4096

LANES = 128
BMG = 256
P_PAD = 2 * N_TOK + E * BMG
NBLK = P_PAD // BMG

NC, NS = 2, 16
NW = NC * NS

_NEG = -1e30


def _gelu_tanh(x):
    return 0.5 * x * (1.0 + jnp.tanh(jnp.sqrt(2.0 / jnp.pi) * (x + 0.044715 * x ** 3)))


_GBM = 512


def _gate_kernel(x_ref, wg_ref, bg_ref, out_ref):
    lane = lax.broadcasted_iota(jnp.int32, (_GBM, LANES), 1)
    logits = (jnp.dot(x_ref[...], wg_ref[...], preferred_element_type=jnp.float32)
              + bg_ref[...]) * (1.0 / jnp.sqrt(jnp.float32(D_IN)))
    logits = jnp.where(lane < E, logits, _NEG)
    m1 = jnp.max(logits, axis=1, keepdims=True)
    p = jnp.exp(logits - m1)
    probs = p / jnp.sum(p, axis=1, keepdims=True)
    i1 = jnp.min(jnp.where(logits >= m1, lane, LANES), axis=1, keepdims=True)
    logits2 = jnp.where(lane == i1, _NEG, logits)
    m2 = jnp.max(logits2, axis=1, keepdims=True)
    i2 = jnp.min(jnp.where(logits2 >= m2, lane, LANES), axis=1, keepdims=True)
    w1 = jnp.sum(probs * (lane == i1), axis=1, keepdims=True)
    w2 = jnp.sum(probs * (lane == i2), axis=1, keepdims=True)
    out = jnp.where(lane == 0, i1.astype(jnp.float32), 0.0)
    out = jnp.where(lane == 1, i2.astype(jnp.float32), out)
    out = jnp.where(lane == 2, w1, out)
    out = jnp.where(lane == 3, w2, out)
    out_ref[...] = out


def _gate(xf, Wg, bg):
    wg_pad = jnp.pad(Wg, ((0, 0), (0, LANES - E)))
    bg_pad = jnp.pad(bg, (0, LANES - E)).reshape(1, LANES)
    return pl.pallas_call(
        _gate_kernel,
        grid=(N_TOK // _GBM,),
        in_specs=[
            pl.BlockSpec((_GBM, D_IN), lambda i: (i, 0)),
            pl.BlockSpec((D_IN, LANES), lambda i: (0, 0)),
            pl.BlockSpec((1, LANES), lambda i: (0, 0)),
        ],
        out_specs=pl.BlockSpec((_GBM, LANES), lambda i: (i, 0)),
        out_shape=jax.ShapeDtypeStruct((N_TOK, LANES), jnp.float32),
    )(xf, wg_pad, bg_pad)


def _metadata(gate_pack):
    i1 = gate_pack[:, 0].astype(jnp.int32)
    i2 = gate_pack[:, 1].astype(jnp.int32)
    wv = gate_pack[:, 2:4]
    e_flat = jnp.stack([i1, i2], axis=1).reshape(-1)
    onehot = (e_flat[:, None] == jnp.arange(E)[None, :]).astype(jnp.int32)
    counts = onehot.sum(axis=0)
    prefix = jnp.cumsum(onehot, axis=0) - onehot
    rank = jnp.take_along_axis(prefix, e_flat[:, None], axis=1)[:, 0]
    padded = ((counts + BMG - 1) // BMG) * BMG
    cpad = jnp.cumsum(padded)
    poff = cpad - padded
    pos = poff[e_flat] + rank
    row_ids = jnp.zeros((P_PAD,), jnp.int32).at[pos].set(
        jnp.arange(2 * N_TOK, dtype=jnp.int32) // 2)
    wrow = jnp.zeros((P_PAD,), jnp.float32).at[pos].set(wv.reshape(-1))
    blk_exp = jnp.searchsorted(
        cpad, jnp.arange(NBLK, dtype=jnp.int32) * BMG, side='right'
    ).astype(jnp.int32)
    nbused = (cpad[-1] // BMG).astype(jnp.int32).reshape(1)
    return row_ids, wrow.reshape(P_PAD, 1), blk_exp, nbused, pos.astype(jnp.int32)


_DCH = 64
_DROWS = P_PAD // NW


def _dispatch_body(ids_hbm, x_hbm, out_hbm, idx_v, rows_v, sem):
    wid = lax.axis_index("s") * NC + lax.axis_index("c")
    base = wid * _DROWS
    for c in range(_DROWS // _DCH):
        off = base + c * _DCH
        pltpu.sync_copy(ids_hbm.at[pl.ds(off, _DCH)], idx_v)
        pltpu.async_copy(x_hbm.at[idx_v], rows_v, sem).wait()
        pltpu.sync_copy(rows_v, out_hbm.at[pl.ds(off, _DCH)])


def _sc_dispatch(row_ids, xf):
    mesh = plsc.VectorSubcoreMesh(
        core_axis_name="c", subcore_axis_name="s", num_cores=NC, num_subcores=NS)
    f = pl.kernel(
        _dispatch_body,
        out_type=jax.ShapeDtypeStruct((P_PAD, D_IN), jnp.float32),
        mesh=mesh,
        scratch_types=[
            pltpu.VMEM((_DCH,), jnp.int32),
            pltpu.VMEM((_DCH, D_IN), jnp.float32),
            pltpu.SemaphoreType.DMA,
        ],
    )
    return f(row_ids, xf)


def _gmm_kernel(blk_exp_ref, nbused_ref, x_ref, w1_ref, b1_ref, w2_ref, b2_ref,
                wrow_ref, out_ref):
    b = pl.program_id(0)

    @pl.when(b < nbused_ref[0])
    def _go():
        xb16 = x_ref[...].astype(jnp.bfloat16)
        w1b = w1_ref[0].astype(jnp.bfloat16)
        h = _gelu_tanh(
            jnp.dot(xb16, w1b, preferred_element_type=jnp.float32) + b1_ref[0])
        w2b = w2_ref[0].astype(jnp.bfloat16)
        y = jnp.dot(h.astype(jnp.bfloat16), w2b,
                    preferred_element_type=jnp.float32) + b2_ref[0]
        out_ref[...] = y * wrow_ref[...]


def _gmm(x_pad, W1, b1, W2, b2, wrow, blk_exp, nbused):
    b1r = b1.reshape(E, 1, D_PROJ)
    b2r = b2.reshape(E, 1, D_OUT)
    grid_spec = pltpu.PrefetchScalarGridSpec(
        num_scalar_prefetch=2,
        grid=(NBLK,),
        in_specs=[
            pl.BlockSpec((BMG, D_IN), lambda b, be, nu: (b, 0)),
            pl.BlockSpec((1, D_IN, D_PROJ), lambda b, be, nu: (be[b], 0, 0)),
            pl.BlockSpec((1, 1, D_PROJ), lambda b, be, nu: (be[b], 0, 0)),
            pl.BlockSpec((1, D_PROJ, D_OUT), lambda b, be, nu: (be[b], 0, 0)),
            pl.BlockSpec((1, 1, D_OUT), lambda b, be, nu: (be[b], 0, 0)),
            pl.BlockSpec((BMG, 1), lambda b, be, nu: (b, 0)),
        ],
        out_specs=pl.BlockSpec((BMG, D_OUT), lambda b, be, nu: (b, 0)),
    )
    return pl.pallas_call(
        _gmm_kernel,
        grid_spec=grid_spec,
        out_shape=jax.ShapeDtypeStruct((P_PAD, D_OUT), jnp.float32),
        compiler_params=pltpu.CompilerParams(
            dimension_semantics=("arbitrary",)),
    )(blk_exp, nbused, x_pad, W1, b1r, W2, b2r, wrow)


_CCH = 32
_CTOK = N_TOK // NW


def _combine_body(pos_hbm, ypw_hbm, y_hbm, idx_v, rows_v, out_v, sem):
    wid = lax.axis_index("s") * NC + lax.axis_index("c")
    tbase = wid * _CTOK
    for c in range(_CTOK // _CCH):
        t0 = tbase + c * _CCH
        pltpu.sync_copy(pos_hbm.at[pl.ds(2 * t0, 2 * _CCH)], idx_v)
        pltpu.async_copy(ypw_hbm.at[idx_v], rows_v, sem).wait()

        def body(i, _):
            for q in range(D_OUT // 16):
                sl = pl.ds(q * 16, 16)
                out_v[i, sl] = rows_v[2 * i, sl] + rows_v[2 * i + 1, sl]
            return 0

        lax.fori_loop(0, _CCH, body, 0)
        pltpu.sync_copy(out_v, y_hbm.at[pl.ds(t0, _CCH)])


def _sc_combine(pos, ypw):
    mesh = plsc.VectorSubcoreMesh(
        core_axis_name="c", subcore_axis_name="s", num_cores=NC, num_subcores=NS)
    f = pl.kernel(
        _combine_body,
        out_type=jax.ShapeDtypeStruct((N_TOK, D_OUT), jnp.float32),
        mesh=mesh,
        scratch_types=[
            pltpu.VMEM((2 * _CCH,), jnp.int32),
            pltpu.VMEM((2 * _CCH, D_OUT), jnp.float32),
            pltpu.VMEM((_CCH, D_OUT), jnp.float32),
            pltpu.SemaphoreType.DMA,
        ],
    )
    return f(pos, ypw)


@jax.jit
def kernel(x, Wg, bg, W1, b1, W2, b2):
    in_shape = x.shape
    xf = x.reshape(-1, D_IN)
    gate_pack = _gate(xf, Wg, bg)
    row_ids, wrow, blk_exp, nbused, pos = _metadata(gate_pack)
    x_pad = _sc_dispatch(row_ids, xf)
    ypw = _gmm(x_pad, W1, b1, W2, b2, wrow, blk_exp, nbused)
    y = _sc_combine(pos, ypw)
    return y.reshape(in_shape[:-1] + (D_OUT,))


# speedup vs baseline: 5.6567x; 5.6567x over previous
import functools

import jax
import jax.numpy as jnp
from jax.experimental import pallas as pl
from jax.experimental.pallas import tpu as pltpu

E = 8
TOP_K = 2
D_IN = 1024
D_OUT = 1024
D_PROJ = 256
N_TOK = 4096

BM = 2048
CH = 256
LANES = 128
D_CAT = E * D_PROJ

_NEG = -1e30


def _gelu_tanh(x):
    return 0.5 * x * (1.0 + jnp.tanh(jnp.sqrt(2.0 / jnp.pi) * (x + 0.044715 * x ** 3)))


def _moe_kernel(x_ref, wg_ref, bg_ref, w1_ref, b1_ref, w2_ref, b2_ref,
                out_ref):
    lane = jax.lax.broadcasted_iota(jnp.int32, (CH, LANES), 1)
    for c in range(BM // CH):
        rows = pl.ds(c * CH, CH)
        xb = x_ref[rows, :]

        logits = (jnp.dot(xb, wg_ref[...], preferred_element_type=jnp.float32)
                  + bg_ref[...]) * (1.0 / jnp.sqrt(jnp.float32(D_IN)))
        logits = jnp.where(lane < E, logits, _NEG)
        m1 = jnp.max(logits, axis=1, keepdims=True)
        p = jnp.exp(logits - m1)
        probs = p / jnp.sum(p, axis=1, keepdims=True)
        i1 = jnp.min(jnp.where(logits >= m1, lane, LANES), axis=1, keepdims=True)
        logits2 = jnp.where(lane == i1, _NEG, logits)
        m2 = jnp.max(logits2, axis=1, keepdims=True)
        i2 = jnp.min(jnp.where(logits2 >= m2, lane, LANES), axis=1, keepdims=True)
        wfull = probs * ((lane == i1) | (lane == i2)).astype(jnp.float32)

        xb16 = xb.astype(jnp.bfloat16)
        cols = []
        for g in range(E):
            hg = (jnp.dot(xb16, w1_ref[g].astype(jnp.bfloat16),
                          preferred_element_type=jnp.float32)
                  + b1_ref[:, g * D_PROJ:(g + 1) * D_PROJ])
            cols.append((_gelu_tanh(hg) * wfull[:, g:g + 1]).astype(jnp.bfloat16))
        g16 = jnp.concatenate(cols, axis=1)
        y = jnp.dot(g16, w2_ref[...].astype(jnp.bfloat16),
                    preferred_element_type=jnp.float32)
        y += jnp.dot(wfull.astype(jnp.bfloat16), b2_ref[...].astype(jnp.bfloat16),
                     preferred_element_type=jnp.float32)
        out_ref[rows, :] = y


@jax.jit
def kernel(x, Wg, bg, W1, b1, W2, b2):
    in_shape = x.shape
    xf = x.reshape(-1, D_IN)
    n = xf.shape[0]
    wg_pad = jnp.pad(Wg, ((0, 0), (0, LANES - E)))
    bg_pad = jnp.pad(bg, (0, LANES - E)).reshape(1, LANES)
    b1_cat = b1.reshape(1, D_CAT)
    w2_stack = W2.reshape(D_CAT, D_OUT)
    b2_pad = jnp.pad(b2, ((0, LANES - E), (0, 0)))
    grid = (n // BM,)
    y = pl.pallas_call(
        _moe_kernel,
        grid=grid,
        in_specs=[
            pl.BlockSpec((BM, D_IN), lambda i: (i, 0)),
            pl.BlockSpec((D_IN, LANES), lambda i: (0, 0)),
            pl.BlockSpec((1, LANES), lambda i: (0, 0)),
            pl.BlockSpec((E, D_IN, D_PROJ), lambda i: (0, 0, 0)),
            pl.BlockSpec((1, D_CAT), lambda i: (0, 0)),
            pl.BlockSpec((D_CAT, D_OUT), lambda i: (0, 0)),
            pl.BlockSpec((LANES, D_OUT), lambda i: (0, 0)),
        ],
        out_specs=pl.BlockSpec((BM, D_OUT), lambda i: (i, 0)),
        out_shape=jax.ShapeDtypeStruct((n, D_OUT), jnp.float32),
        compiler_params=pltpu.CompilerParams(
            dimension_semantics=("parallel",)),
    )(xf, wg_pad, bg_pad, W1, b1_cat, w2_stack, b2_pad)
    return y.reshape(in_shape[:-1] + (D_OUT,))
